# baseline (device time: 311797 ns/iter reference)
import jax
import jax.numpy as jnp
from jax import lax
from jax.experimental import pallas as pl
from jax.experimental.pallas import tpu as pltpu

N_DEV = 4
M = 4096
N = 2048
M_CHUNK = M // N_DEV
HALF = M_CHUNK // 2
K = 2
SUB = HALF // K
N_STEP = N_DEV - 1
DIRS = (0, 1)


def kernel(x):

    def body(x_hbm, out_ref, stage_cw, stage_ccw, fetch_sems,
             rs_send, rs_recv, ag_send, ag_recv):
        my = lax.axis_index("i")
        left = lax.rem(my - 1 + N_DEV, N_DEV)
        right = lax.rem(my + 1, N_DEV)


        def peer(d):
            return right if d == 0 else left

        def sc(d, s):
            off = -s if d == 0 else s
            return lax.rem(my + off + N_DEV, N_DEV)

        def rc(d, s):
            off = -s - 1 if d == 0 else s + 1
            return lax.rem(my + off + N_DEV, N_DEV)

        def ac(d, s):
            off = 1 - s if d == 0 else s - 1
            return lax.rem(my + off + N_DEV, N_DEV)

        def subrow(c, d, j):
            return pl.ds(c * M_CHUNK + d * HALF + j * SUB, SUB)

        def rs_desc(d, s, j):
            base = x_hbm.at[0] if s == 0 else out_ref
            c = sc(d, s)
            return pltpu.make_async_remote_copy(
                src_ref=base.at[subrow(c, d, j), :],
                dst_ref=out_ref.at[subrow(c, d, j), :],
                send_sem=rs_send.at[d, s, j],
                recv_sem=rs_recv.at[d, s, j],
                device_id=(peer(d),),
                device_id_type=pl.DeviceIdType.MESH,
            )

        def ag_desc(d, s, j):
            c = ac(d, s)
            return pltpu.make_async_remote_copy(
                src_ref=out_ref.at[subrow(c, d, j), :],
                dst_ref=out_ref.at[subrow(c, d, j), :],
                send_sem=ag_send.at[d, s, j],
                recv_sem=ag_recv.at[d, s, j],
                device_id=(peer(d),),
                device_id_type=pl.DeviceIdType.MESH,
            )

        stages = (stage_cw, stage_ccw)

        def start_fetch(s):
            fs = []
            for d in DIRS:
                f = pltpu.make_async_copy(
                    x_hbm.at[0, pl.ds(rc(d, s) * M_CHUNK + d * HALF, HALF), :],
                    stages[d],
                    fetch_sems.at[d],
                )
                f.start()
                fs.append(f)
            return fs

        rs_inflight = {}
        for d in DIRS:
            for j in range(K):
                r = rs_desc(d, 0, j)
                r.start()
                rs_inflight[(d, 0, j)] = r
        fetches = start_fetch(0)

        ag_inflight = {}
        for s in range(N_STEP):
            for j in range(K):
                for d in DIRS:
                    rs_inflight[(d, s, j)].wait()
                    if j == 0:
                        fetches[d].wait()
                    c = rc(d, s)
                    out_ref[subrow(c, d, j), :] = (
                        out_ref[subrow(c, d, j), :]
                        + stages[d][pl.ds(j * SUB, SUB), :]
                    )
                    if s < N_STEP - 1:
                        r = rs_desc(d, s + 1, j)
                        r.start()
                        rs_inflight[(d, s + 1, j)] = r
                    else:
                        a = ag_desc(d, 0, j)
                        a.start()
                        ag_inflight[(d, 0, j)] = a
            if s < N_STEP - 1:
                fetches = start_fetch(s + 1)

        for s in range(N_STEP):
            for j in range(K):
                for d in DIRS:
                    ag_inflight[(d, s, j)].wait()
                    if s < N_STEP - 1:
                        a = ag_desc(d, s + 1, j)
                        a.start()
                        ag_inflight[(d, s + 1, j)] = a

    return pl.pallas_call(
        body,
        out_shape=jax.ShapeDtypeStruct((M, N), jnp.float32),
        in_specs=[pl.BlockSpec(memory_space=pl.ANY)],
        out_specs=pl.BlockSpec(memory_space=pltpu.VMEM),
        scratch_shapes=[
            pltpu.VMEM((HALF, N), jnp.float32),
            pltpu.VMEM((HALF, N), jnp.float32),
            pltpu.SemaphoreType.DMA((2,)),
            pltpu.SemaphoreType.DMA((2, N_STEP, K)),
            pltpu.SemaphoreType.DMA((2, N_STEP, K)),
            pltpu.SemaphoreType.DMA((2, N_STEP, K)),
            pltpu.SemaphoreType.DMA((2, N_STEP, K)),
        ],
        compiler_params=pltpu.CompilerParams(
            vmem_limit_bytes=56 * 1024 * 1024,
        ),
    )(x)


# device time: 308628 ns/iter; 1.0103x vs baseline; 1.0103x over previous
import jax
import jax.numpy as jnp
from jax import lax
from jax.experimental import pallas as pl
from jax.experimental.pallas import tpu as pltpu

N_DEV = 4
M = 4096
N = 2048
M_CHUNK = M // N_DEV
HALF = M_CHUNK // 2
K = 4
SUB = HALF // K
N_STEP = N_DEV - 1
DIRS = (0, 1)


def kernel(x):

    def body(x_hbm, out_ref, stage_cw, stage_ccw, fetch_sems,
             rs_send, rs_recv, ag_send, ag_recv):
        my = lax.axis_index("i")
        left = lax.rem(my - 1 + N_DEV, N_DEV)
        right = lax.rem(my + 1, N_DEV)

        barrier_sem = pltpu.get_barrier_semaphore()
        for nbr in (left, right):
            pl.semaphore_signal(
                barrier_sem, inc=1,
                device_id=(nbr,), device_id_type=pl.DeviceIdType.MESH,
            )
        pl.semaphore_wait(barrier_sem, 2)

        def peer(d):
            return right if d == 0 else left

        def sc(d, s):
            off = -s if d == 0 else s
            return lax.rem(my + off + N_DEV, N_DEV)

        def rc(d, s):
            off = -s - 1 if d == 0 else s + 1
            return lax.rem(my + off + N_DEV, N_DEV)

        def ac(d, s):
            off = 1 - s if d == 0 else s - 1
            return lax.rem(my + off + N_DEV, N_DEV)

        def subrow(c, d, j):
            return pl.ds(c * M_CHUNK + d * HALF + j * SUB, SUB)

        def rs_desc(d, s, j):
            base = x_hbm.at[0] if s == 0 else out_ref
            c = sc(d, s)
            return pltpu.make_async_remote_copy(
                src_ref=base.at[subrow(c, d, j), :],
                dst_ref=out_ref.at[subrow(c, d, j), :],
                send_sem=rs_send.at[d, s, j],
                recv_sem=rs_recv.at[d, s, j],
                device_id=(peer(d),),
                device_id_type=pl.DeviceIdType.MESH,
            )

        def ag_desc(d, s, j):
            c = ac(d, s)
            return pltpu.make_async_remote_copy(
                src_ref=out_ref.at[subrow(c, d, j), :],
                dst_ref=out_ref.at[subrow(c, d, j), :],
                send_sem=ag_send.at[d, s, j],
                recv_sem=ag_recv.at[d, s, j],
                device_id=(peer(d),),
                device_id_type=pl.DeviceIdType.MESH,
            )

        stages = (stage_cw, stage_ccw)

        def start_fetch(s):
            fs = []
            for d in DIRS:
                f = pltpu.make_async_copy(
                    x_hbm.at[0, pl.ds(rc(d, s) * M_CHUNK + d * HALF, HALF), :],
                    stages[d],
                    fetch_sems.at[d],
                )
                f.start()
                fs.append(f)
            return fs

        rs_inflight = {}
        for d in DIRS:
            for j in range(K):
                r = rs_desc(d, 0, j)
                r.start()
                rs_inflight[(d, 0, j)] = r
        fetches = start_fetch(0)

        ag_inflight = {}
        for s in range(N_STEP):
            for j in range(K):
                for d in DIRS:
                    rs_inflight[(d, s, j)].wait()
                    if j == 0:
                        fetches[d].wait()
                    c = rc(d, s)
                    out_ref[subrow(c, d, j), :] = (
                        out_ref[subrow(c, d, j), :]
                        + stages[d][pl.ds(j * SUB, SUB), :]
                    )
                    if s < N_STEP - 1:
                        r = rs_desc(d, s + 1, j)
                        r.start()
                        rs_inflight[(d, s + 1, j)] = r
                    else:
                        a = ag_desc(d, 0, j)
                        a.start()
                        ag_inflight[(d, 0, j)] = a
            if s < N_STEP - 1:
                fetches = start_fetch(s + 1)

        for s in range(N_STEP):
            for j in range(K):
                for d in DIRS:
                    ag_inflight[(d, s, j)].wait()
                    if s < N_STEP - 1:
                        a = ag_desc(d, s + 1, j)
                        a.start()
                        ag_inflight[(d, s + 1, j)] = a

    return pl.pallas_call(
        body,
        out_shape=jax.ShapeDtypeStruct((M, N), jnp.float32),
        in_specs=[pl.BlockSpec(memory_space=pl.ANY)],
        out_specs=pl.BlockSpec(memory_space=pltpu.VMEM),
        scratch_shapes=[
            pltpu.VMEM((HALF, N), jnp.float32),
            pltpu.VMEM((HALF, N), jnp.float32),
            pltpu.SemaphoreType.DMA((2,)),
            pltpu.SemaphoreType.DMA((2, N_STEP, K)),
            pltpu.SemaphoreType.DMA((2, N_STEP, K)),
            pltpu.SemaphoreType.DMA((2, N_STEP, K)),
            pltpu.SemaphoreType.DMA((2, N_STEP, K)),
        ],
        compiler_params=pltpu.CompilerParams(
            collective_id=0,
            vmem_limit_bytes=56 * 1024 * 1024,
        ),
    )(x)
